# Initial kernel scaffold; baseline (speedup 1.0000x reference)
import jax
import jax.numpy as jnp
from jax.experimental import pallas as pl


def kernel(z, W):
    raise NotImplementedError("write your pallas kernel here")


# speedup vs baseline: 2.3145x; 2.3145x over previous
import functools

import jax
import jax.numpy as jnp
from jax.experimental import pallas as pl
from jax.experimental.pallas import tpu as pltpu

BETA = 0.5
BLOCK = 1024


def _vq_block_kernel(z_ref, w_ref, zq_ref, loss_ref, *, G, K, C):
    @pl.when(pl.program_id(0) == 0)
    def _init():
        loss_ref[...] = jnp.zeros((1, 1), jnp.float32)

    total = jnp.float32(0.0)
    for g in range(G):
        zi = z_ref[:, g * C:(g + 1) * C]
        w = w_ref[g]
        zi2 = zi * zi
        t = zi2[:, :8] + zi2[:, 8:]
        t = t[:, :4] + t[:, 4:]
        t = t[:, :2] + t[:, 2:]
        s_z = t[:, :1] + t[:, 1:]
        d = (s_z
             + jnp.sum(w ** 2, axis=1)
             - 2.0 * jnp.matmul(zi, w.T))
        d_min = jnp.min(d, axis=1, keepdims=True)
        iota_k = jax.lax.broadcasted_iota(jnp.int32, d.shape, 1)
        idx = jnp.min(jnp.where(d == d_min, iota_k, K), axis=1, keepdims=True)
        one_hot = (iota_k == idx).astype(jnp.float32)
        zq = jnp.matmul(one_hot, w)
        diff = zq - zi
        zq_ref[:, g * C:(g + 1) * C] = zi + diff
        total = total + jnp.sum(diff * diff)
    loss_ref[...] = loss_ref[...] + total


def kernel(z, W):
    N, D = z.shape
    G, K, C = W.shape
    grid = N // BLOCK
    zq, loss_sum = pl.pallas_call(
        functools.partial(_vq_block_kernel, G=G, K=K, C=C),
        grid=(grid,),
        in_specs=[
            pl.BlockSpec((BLOCK, D), lambda i: (i, 0)),
            pl.BlockSpec((G, K, C), lambda i: (0, 0, 0)),
        ],
        out_specs=[
            pl.BlockSpec((BLOCK, D), lambda i: (i, 0)),
            pl.BlockSpec((1, 1), lambda i: (0, 0)),
        ],
        out_shape=[
            jax.ShapeDtypeStruct((N, D), jnp.float32),
            jax.ShapeDtypeStruct((1, 1), jnp.float32),
        ],
        compiler_params=pltpu.CompilerParams(
            dimension_semantics=("arbitrary",),
        ),
    )(z, W)
    loss = (1.0 + BETA) * loss_sum[0, 0] / (N * D)
    return (zq, loss)


# speedup vs baseline: 2.5304x; 1.0933x over previous
import functools

import jax
import jax.numpy as jnp
from jax import lax
from jax.experimental import pallas as pl
from jax.experimental.pallas import tpu as pltpu
from jax.experimental.pallas import tpu_sc as plsc

BETA = 0.5
BLOCK = 2048


def _vq_block_kernel(z_ref, w_ref, zq_ref, loss_ref, *, G, K, C):
    @pl.when(pl.program_id(0) == 0)
    def _init():
        loss_ref[...] = jnp.zeros((1, 1), jnp.float32)

    total = jnp.float32(0.0)
    for g in range(G):
        zi = z_ref[:, g * C:(g + 1) * C]
        w = w_ref[g]
        zi2 = zi * zi
        t = zi2[:, :8] + zi2[:, 8:]
        t = t[:, :4] + t[:, 4:]
        t = t[:, :2] + t[:, 2:]
        s_z = t[:, :1] + t[:, 1:]
        d = (s_z
             + jnp.sum(w ** 2, axis=1)
             - 2.0 * jnp.matmul(zi, w.T))
        d_min = jnp.min(d, axis=1, keepdims=True)
        iota_k = jax.lax.broadcasted_iota(jnp.int32, d.shape, 1)
        idx = jnp.min(jnp.where(d == d_min, iota_k, K), axis=1, keepdims=True)
        one_hot = (iota_k == idx).astype(jnp.float32)
        zq = jnp.matmul(one_hot, w)
        diff = zq - zi
        zq_ref[:, g * C:(g + 1) * C] = zi + diff
        total = total + jnp.sum(diff * diff)
    loss_ref[...] = loss_ref[...] + total


def _gather_rows(table, idx_flat, n_rows, C):
    info = plsc.get_sparse_core_info()
    nw = info.num_cores * info.num_subcores
    nc = info.num_cores
    per_w = n_rows // nw
    chunk = 2048
    n_chunks = per_w // chunk
    mesh = plsc.VectorSubcoreMesh(core_axis_name="c", subcore_axis_name="s")

    @functools.partial(
        pl.kernel,
        mesh=mesh,
        out_type=jax.ShapeDtypeStruct((n_rows, C), jnp.float32),
        scratch_types=[
            pltpu.VMEM((chunk,), jnp.int32),
            pltpu.VMEM((chunk, C), jnp.float32),
            pltpu.SemaphoreType.DMA,
        ],
    )
    def gather_k(table_hbm, idx_hbm, out_hbm, idx_v, rows_v, sem):
        wid = lax.axis_index("s") * nc + lax.axis_index("c")
        for j in range(n_chunks):
            base = wid * per_w + j * chunk
            pltpu.sync_copy(idx_hbm.at[pl.ds(base, chunk)], idx_v)
            pltpu.async_copy(table_hbm.at[idx_v], rows_v, sem).wait()
            pltpu.sync_copy(rows_v, out_hbm.at[pl.ds(base, chunk)])

    return gather_k(table, idx_flat)


def kernel(z, W):
    N, D = z.shape
    G, K, C = W.shape
    grid = N // BLOCK
    zq, loss_sum = pl.pallas_call(
        functools.partial(_vq_block_kernel, G=G, K=K, C=C),
        grid=(grid,),
        in_specs=[
            pl.BlockSpec((BLOCK, D), lambda i: (i, 0)),
            pl.BlockSpec((G, K, C), lambda i: (0, 0, 0)),
        ],
        out_specs=[
            pl.BlockSpec((BLOCK, D), lambda i: (i, 0)),
            pl.BlockSpec((1, 1), lambda i: (0, 0)),
        ],
        out_shape=[
            jax.ShapeDtypeStruct((N, D), jnp.float32),
            jax.ShapeDtypeStruct((1, 1), jnp.float32),
        ],
        compiler_params=pltpu.CompilerParams(
            dimension_semantics=("arbitrary",),
        ),
    )(z, W)
    loss = (1.0 + BETA) * loss_sum[0, 0] / (N * D)
    return (zq, loss)


# speedup vs baseline: 3.2842x; 1.2979x over previous
import functools

import jax
import jax.numpy as jnp
from jax import lax
from jax.experimental import pallas as pl
from jax.experimental.pallas import tpu as pltpu
from jax.experimental.pallas import tpu_sc as plsc

BETA = 0.5
BLOCK = 2048


def _dist_kernel(z_ref, w_ref, idx_ref, loss_ref, *, G, K, C):
    @pl.when(pl.program_id(0) == 0)
    def _init():
        loss_ref[...] = jnp.zeros((1, 1), jnp.float32)

    total = jnp.float32(0.0)
    cols = []
    for g in range(G):
        zi = z_ref[:, g * C:(g + 1) * C]
        w = w_ref[g]
        zi2 = zi * zi
        t = zi2[:, :8] + zi2[:, 8:]
        t = t[:, :4] + t[:, 4:]
        t = t[:, :2] + t[:, 2:]
        s_z = t[:, :1] + t[:, 1:]
        d = (s_z
             + jnp.sum(w ** 2, axis=1)
             - 2.0 * jnp.matmul(zi, w.T))
        d_min = jnp.min(d, axis=1, keepdims=True)
        iota_k = jax.lax.broadcasted_iota(jnp.int32, d.shape, 1)
        idx = jnp.min(jnp.where(d == d_min, iota_k, K), axis=1, keepdims=True)
        cols.append(idx + g * K)
        total = total + jnp.sum(d_min)
    idx_ref[...] = jnp.concatenate(cols, axis=1)
    loss_ref[...] = loss_ref[...] + total


def _sc_gather(table_flat, idx_flat, N, D, G, C):
    info = plsc.get_sparse_core_info()
    nw = info.num_cores * info.num_subcores
    nc = info.num_cores
    rows_w = N // nw
    R = 128
    n_chunks = rows_w // R
    tab_len = table_flat.shape[0]
    mesh = plsc.VectorSubcoreMesh(core_axis_name="c", subcore_axis_name="s")

    @functools.partial(
        pl.kernel,
        mesh=mesh,
        out_type=jax.ShapeDtypeStruct((N, D), jnp.float32),
        scratch_types=[
            pltpu.VMEM((tab_len,), jnp.float32),
            pltpu.VMEM((rows_w * G,), jnp.int32),
            pltpu.VMEM((R, D), jnp.float32),
        ],
        compiler_params=pltpu.CompilerParams(needs_layout_passes=False),
    )
    def gather_k(table_hbm, idx_hbm, out_hbm, table_v, idx_v, out_v):
        wid = lax.axis_index("s") * nc + lax.axis_index("c")
        pltpu.sync_copy(table_hbm, table_v)
        pltpu.sync_copy(idx_hbm.at[pl.ds(wid * (rows_w * G), rows_w * G)],
                        idx_v)
        iota = lax.iota(jnp.int32, 16)
        iota_g = iota * G

        def chunk_body(ch, carry):
            def j_body(j, carry2):
                rbase = j * 16
                rows = rbase + iota
                for g in range(G):
                    pos = iota_g + ((ch * R + rbase) * G + g)
                    fk = plsc.load_gather(idx_v, [pos])
                    addr = fk * C
                    for c in range(C):
                        vals = plsc.load_gather(table_v, [addr + c])
                        col = jnp.full((16,), g * C + c, jnp.int32)
                        plsc.store_scatter(out_v, [rows, col], vals)
                return carry2

            lax.fori_loop(0, R // 16, j_body, 0)
            pltpu.sync_copy(out_v,
                            out_hbm.at[pl.ds(wid * rows_w + ch * R, R)])
            return carry

        lax.fori_loop(0, n_chunks, chunk_body, 0)

    return gather_k(table_flat, idx_flat)


def kernel(z, W):
    N, D = z.shape
    G, K, C = W.shape
    grid = N // BLOCK
    idx, loss_sum = pl.pallas_call(
        functools.partial(_dist_kernel, G=G, K=K, C=C),
        grid=(grid,),
        in_specs=[
            pl.BlockSpec((BLOCK, D), lambda i: (i, 0)),
            pl.BlockSpec((G, K, C), lambda i: (0, 0, 0)),
        ],
        out_specs=[
            pl.BlockSpec((BLOCK, G), lambda i: (i, 0)),
            pl.BlockSpec((1, 1), lambda i: (0, 0)),
        ],
        out_shape=[
            jax.ShapeDtypeStruct((N, G), jnp.int32),
            jax.ShapeDtypeStruct((1, 1), jnp.float32),
        ],
        compiler_params=pltpu.CompilerParams(
            dimension_semantics=("arbitrary",),
        ),
    )(z, W)
    zq = _sc_gather(W.reshape(G * K * C), idx.reshape(N * G), N, D, G, C)
    loss = (1.0 + BETA) * loss_sum[0, 0] / (N * D)
    return (zq, loss)


# speedup vs baseline: 3.7035x; 1.1277x over previous
import functools

import jax
import jax.numpy as jnp
from jax import lax
from jax.experimental import pallas as pl
from jax.experimental.pallas import tpu as pltpu
from jax.experimental.pallas import tpu_sc as plsc

BETA = 0.5
BLOCK = 2048


def _dist_kernel(z_ref, w_ref, idx_ref, loss_ref, *, G, K, C):
    @pl.when(pl.program_id(0) == 0)
    def _init():
        loss_ref[...] = jnp.zeros((1, 1), jnp.float32)

    total = jnp.float32(0.0)
    cols = []
    for g in range(G):
        zi = z_ref[:, g * C:(g + 1) * C]
        w = w_ref[g]
        zi2 = zi * zi
        t = zi2[:, :8] + zi2[:, 8:]
        t = t[:, :4] + t[:, 4:]
        t = t[:, :2] + t[:, 2:]
        s_z = t[:, :1] + t[:, 1:]
        d = (s_z
             + jnp.sum(w ** 2, axis=1)
             - 2.0 * jnp.matmul(zi, w.T))
        d_min = jnp.min(d, axis=1, keepdims=True)
        iota_k = jax.lax.broadcasted_iota(jnp.int32, d.shape, 1)
        idx = jnp.min(jnp.where(d == d_min, iota_k, K), axis=1, keepdims=True)
        cols.append(idx + g * K)
        total = total + jnp.sum(d_min)
    idx_ref[...] = jnp.concatenate(cols, axis=1)
    loss_ref[...] = loss_ref[...] + total


def _sc_gather(table_flat, idx_flat, N, D, G, C):
    info = plsc.get_sparse_core_info()
    nw = info.num_cores * info.num_subcores
    nc = info.num_cores
    rows_w = N // nw
    R = 256
    n_chunks = rows_w // R
    tab_len = table_flat.shape[0]
    mesh = plsc.VectorSubcoreMesh(core_axis_name="c", subcore_axis_name="s")

    @functools.partial(
        pl.kernel,
        mesh=mesh,
        out_type=jax.ShapeDtypeStruct((N, D), jnp.float32),
        scratch_types=[
            pltpu.VMEM((tab_len,), jnp.float32),
            pltpu.VMEM((rows_w * G,), jnp.int32),
            pltpu.VMEM((R, D), jnp.float32),
        ],
        compiler_params=pltpu.CompilerParams(needs_layout_passes=False),
    )
    def gather_k(table_hbm, idx_hbm, out_hbm, table_v, idx_v, out_v):
        wid = lax.axis_index("s") * nc + lax.axis_index("c")
        pltpu.sync_copy(table_hbm, table_v)
        pltpu.sync_copy(idx_hbm.at[pl.ds(wid * (rows_w * G), rows_w * G)],
                        idx_v)
        iota = lax.iota(jnp.int32, 16)
        iota_g = iota * G

        def chunk_body(ch, carry):
            @plsc.parallel_loop(0, R // 16, unroll=2)
            def j_body(j):
                rbase = j * 16
                rows = rbase + iota
                for g in range(G):
                    pos = iota_g + ((ch * R + rbase) * G + g)
                    fk = plsc.load_gather(idx_v, [pos])
                    addr = fk * C
                    vals = [plsc.load_gather(table_v, [addr + c])
                            for c in range(C)]
                    for c in range(C):
                        col = jnp.full((16,), g * C + c, jnp.int32)
                        plsc.store_scatter(out_v, [rows, col], vals[c])

            pltpu.sync_copy(out_v,
                            out_hbm.at[pl.ds(wid * rows_w + ch * R, R)])
            return carry

        lax.fori_loop(0, n_chunks, chunk_body, 0)

    return gather_k(table_flat, idx_flat)


def kernel(z, W):
    N, D = z.shape
    G, K, C = W.shape
    grid = N // BLOCK
    idx, loss_sum = pl.pallas_call(
        functools.partial(_dist_kernel, G=G, K=K, C=C),
        grid=(grid,),
        in_specs=[
            pl.BlockSpec((BLOCK, D), lambda i: (i, 0)),
            pl.BlockSpec((G, K, C), lambda i: (0, 0, 0)),
        ],
        out_specs=[
            pl.BlockSpec((BLOCK, G), lambda i: (i, 0)),
            pl.BlockSpec((1, 1), lambda i: (0, 0)),
        ],
        out_shape=[
            jax.ShapeDtypeStruct((N, G), jnp.int32),
            jax.ShapeDtypeStruct((1, 1), jnp.float32),
        ],
        compiler_params=pltpu.CompilerParams(
            dimension_semantics=("arbitrary",),
        ),
    )(z, W)
    zq = _sc_gather(W.reshape(G * K * C), idx.reshape(N * G), N, D, G, C)
    loss = (1.0 + BETA) * loss_sum[0, 0] / (N * D)
    return (zq, loss)


# speedup vs baseline: 4.5745x; 1.2352x over previous
import functools

import jax
import jax.numpy as jnp
from jax import lax
from jax.experimental import pallas as pl
from jax.experimental.pallas import tpu as pltpu
from jax.experimental.pallas import tpu_sc as plsc

BETA = 0.5
BLOCK = 2048


def _dist_kernel(z_ref, w_ref, idx_ref, loss_ref, *, G, K, C):
    @pl.when(pl.program_id(0) == 0)
    def _init():
        loss_ref[...] = jnp.zeros((1, 1), jnp.float32)

    B = z_ref.shape[0]
    sz_cols = []
    for g in range(G):
        zi = z_ref[:, g * C:(g + 1) * C]
        zi2 = zi * zi
        t = zi2[:, :8] + zi2[:, 8:]
        t = t[:, :4] + t[:, 4:]
        t = t[:, :2] + t[:, 2:]
        sz_cols.append(t[:, :1] + t[:, 1:])
    s_z_t = jnp.transpose(jnp.concatenate(sz_cols, axis=1))

    total = jnp.float32(0.0)
    rows = []
    for g in range(G):
        zi = z_ref[:, g * C:(g + 1) * C]
        w = w_ref[g]
        m2 = jax.lax.dot_general(w + w, zi, (((1,), (1,)), ((), ())))
        d = (s_z_t[g:g + 1, :]
             + jnp.sum(w ** 2, axis=1, keepdims=True)
             - m2)
        d_min = jnp.min(d, axis=0, keepdims=True)
        iota_k = jax.lax.broadcasted_iota(jnp.int32, d.shape, 0)
        idx = jnp.min(jnp.where(d == d_min, iota_k, K), axis=0, keepdims=True)
        rows.append(idx + g * K)
        total = total + jnp.sum(d_min)
    idx_ref[...] = jnp.concatenate(rows, axis=0)
    loss_ref[...] = loss_ref[...] + total


def _sc_gather(table_flat, idx_flat, N, D, G, C):
    info = plsc.get_sparse_core_info()
    nw = info.num_cores * info.num_subcores
    nc = info.num_cores
    rows_w = N // nw
    R = 256
    n_chunks = rows_w // R
    tab_len = table_flat.shape[0]
    mesh = plsc.VectorSubcoreMesh(core_axis_name="c", subcore_axis_name="s")

    @functools.partial(
        pl.kernel,
        mesh=mesh,
        out_type=jax.ShapeDtypeStruct((N, D), jnp.float32),
        scratch_types=[
            pltpu.VMEM((tab_len,), jnp.float32),
            pltpu.VMEM((rows_w * G,), jnp.int32),
            pltpu.VMEM((R, D), jnp.float32),
        ],
        compiler_params=pltpu.CompilerParams(needs_layout_passes=False),
    )
    def gather_k(table_hbm, idx_hbm, out_hbm, table_v, idx_v, out_v):
        wid = lax.axis_index("s") * nc + lax.axis_index("c")
        pltpu.sync_copy(table_hbm, table_v)
        for g in range(G):
            pltpu.sync_copy(idx_hbm.at[pl.ds(g * N + wid * rows_w, rows_w)],
                            idx_v.at[pl.ds(g * rows_w, rows_w)])
        iota = lax.iota(jnp.int32, 16)

        def chunk_body(ch, carry):
            @plsc.parallel_loop(0, R // 16, unroll=2)
            def j_body(j):
                rbase = j * 16
                rows = rbase + iota
                for g in range(G):
                    fk = idx_v[pl.ds(g * rows_w + ch * R + rbase, 16)]
                    addr = fk * C
                    vals = [plsc.load_gather(table_v, [addr + c])
                            for c in range(C)]
                    for c in range(C):
                        col = jnp.full((16,), g * C + c, jnp.int32)
                        plsc.store_scatter(out_v, [rows, col], vals[c])

            pltpu.sync_copy(out_v,
                            out_hbm.at[pl.ds(wid * rows_w + ch * R, R)])
            return carry

        lax.fori_loop(0, n_chunks, chunk_body, 0)

    return gather_k(table_flat, idx_flat)


def kernel(z, W):
    N, D = z.shape
    G, K, C = W.shape
    grid = N // BLOCK
    idx, loss_sum = pl.pallas_call(
        functools.partial(_dist_kernel, G=G, K=K, C=C),
        grid=(grid,),
        in_specs=[
            pl.BlockSpec((BLOCK, D), lambda i: (i, 0)),
            pl.BlockSpec((G, K, C), lambda i: (0, 0, 0)),
        ],
        out_specs=[
            pl.BlockSpec((G, BLOCK), lambda i: (0, i)),
            pl.BlockSpec((1, 1), lambda i: (0, 0)),
        ],
        out_shape=[
            jax.ShapeDtypeStruct((G, N), jnp.int32),
            jax.ShapeDtypeStruct((1, 1), jnp.float32),
        ],
        compiler_params=pltpu.CompilerParams(
            dimension_semantics=("arbitrary",),
        ),
    )(z, W)
    zq = _sc_gather(W.reshape(G * K * C), idx.reshape(G * N), N, D, G, C)
    loss = (1.0 + BETA) * loss_sum[0, 0] / (N * D)
    return (zq, loss)


# speedup vs baseline: 4.5754x; 1.0002x over previous
import functools

import jax
import jax.numpy as jnp
from jax import lax
from jax.experimental import pallas as pl
from jax.experimental.pallas import tpu as pltpu
from jax.experimental.pallas import tpu_sc as plsc

BETA = 0.5
BLOCK = 2048


def _dist_kernel(z_ref, w_ref, idx_ref, loss_ref, *, G, K, C):
    @pl.when(pl.program_id(0) == 0)
    def _init():
        loss_ref[...] = jnp.zeros((1, 1), jnp.float32)

    B = z_ref.shape[0]
    sz_cols = []
    for g in range(G):
        zi = z_ref[:, g * C:(g + 1) * C]
        zi2 = zi * zi
        t = zi2[:, :8] + zi2[:, 8:]
        t = t[:, :4] + t[:, 4:]
        t = t[:, :2] + t[:, 2:]
        sz_cols.append(t[:, :1] + t[:, 1:])
    s_z_t = jnp.transpose(jnp.concatenate(sz_cols, axis=1))

    total = jnp.float32(0.0)
    rows = []
    for g in range(G):
        zi = z_ref[:, g * C:(g + 1) * C]
        w = w_ref[g]
        m2 = jax.lax.dot_general(w + w, zi, (((1,), (1,)), ((), ())))
        d = (s_z_t[g:g + 1, :]
             + jnp.sum(w ** 2, axis=1, keepdims=True)
             - m2)
        d_min = jnp.min(d, axis=0, keepdims=True)
        iota_k = jax.lax.broadcasted_iota(jnp.int32, d.shape, 0)
        idx = jnp.min(jnp.where(d == d_min, iota_k, K), axis=0, keepdims=True)
        rows.append(idx + g * K)
        total = total + jnp.sum(d_min)
    idx_ref[...] = jnp.concatenate(rows, axis=0)
    loss_ref[...] = loss_ref[...] + total


def _sc_gather(table_flat, idx_flat, N, D, G, C):
    info = plsc.get_sparse_core_info()
    nw = info.num_cores * info.num_subcores
    nc = info.num_cores
    rows_w = N // nw
    R = 256
    n_chunks = rows_w // R
    tab_len = table_flat.shape[0]
    mesh = plsc.VectorSubcoreMesh(core_axis_name="c", subcore_axis_name="s")

    @functools.partial(
        pl.kernel,
        mesh=mesh,
        out_type=jax.ShapeDtypeStruct((N, D), jnp.float32),
        scratch_types=[
            pltpu.VMEM((tab_len,), jnp.float32),
            pltpu.VMEM((rows_w * G,), jnp.int32),
            pltpu.VMEM((R, D), jnp.float32),
        ],
        compiler_params=pltpu.CompilerParams(needs_layout_passes=False),
    )
    def gather_k(table_hbm, idx_hbm, out_hbm, table_v, idx_v, out_v):
        wid = lax.axis_index("s") * nc + lax.axis_index("c")
        pltpu.sync_copy(table_hbm, table_v)
        for g in range(G):
            pltpu.sync_copy(idx_hbm.at[pl.ds(g * N + wid * rows_w, rows_w)],
                            idx_v.at[pl.ds(g * rows_w, rows_w)])
        iota = lax.iota(jnp.int32, 16)

        def chunk_body(ch, carry):
            @plsc.parallel_loop(0, R // 16, unroll=4)
            def j_body(j):
                rbase = j * 16
                rows = rbase + iota
                for g in range(G):
                    fk = idx_v[pl.ds(g * rows_w + ch * R + rbase, 16)]
                    addr = fk * C
                    vals = [plsc.load_gather(table_v, [addr + c])
                            for c in range(C)]
                    for c in range(C):
                        col = jnp.full((16,), g * C + c, jnp.int32)
                        plsc.store_scatter(out_v, [rows, col], vals[c])

            pltpu.sync_copy(out_v,
                            out_hbm.at[pl.ds(wid * rows_w + ch * R, R)])
            return carry

        lax.fori_loop(0, n_chunks, chunk_body, 0)

    return gather_k(table_flat, idx_flat)


def kernel(z, W):
    N, D = z.shape
    G, K, C = W.shape
    grid = N // BLOCK
    idx, loss_sum = pl.pallas_call(
        functools.partial(_dist_kernel, G=G, K=K, C=C),
        grid=(grid,),
        in_specs=[
            pl.BlockSpec((BLOCK, D), lambda i: (i, 0)),
            pl.BlockSpec((G, K, C), lambda i: (0, 0, 0)),
        ],
        out_specs=[
            pl.BlockSpec((G, BLOCK), lambda i: (0, i)),
            pl.BlockSpec((1, 1), lambda i: (0, 0)),
        ],
        out_shape=[
            jax.ShapeDtypeStruct((G, N), jnp.int32),
            jax.ShapeDtypeStruct((1, 1), jnp.float32),
        ],
        compiler_params=pltpu.CompilerParams(
            dimension_semantics=("arbitrary",),
        ),
    )(z, W)
    zq = _sc_gather(W.reshape(G * K * C), idx.reshape(G * N), N, D, G, C)
    loss = (1.0 + BETA) * loss_sum[0, 0] / (N * D)
    return (zq, loss)


# speedup vs baseline: 4.9559x; 1.0831x over previous
import functools

import jax
import jax.numpy as jnp
from jax import lax
from jax.experimental import pallas as pl
from jax.experimental.pallas import tpu as pltpu
from jax.experimental.pallas import tpu_sc as plsc

BETA = 0.5
BLOCK = 2048


def _dist_kernel(z_ref, w_ref, idx_ref, loss_ref, *, G, K, C):
    @pl.when(pl.program_id(0) == 0)
    def _init():
        loss_ref[...] = jnp.zeros((1, 1), jnp.float32)

    B = z_ref.shape[0]
    sz_cols = []
    for g in range(G):
        zi = z_ref[:, g * C:(g + 1) * C]
        zi2 = zi * zi
        t = zi2[:, :8] + zi2[:, 8:]
        t = t[:, :4] + t[:, 4:]
        t = t[:, :2] + t[:, 2:]
        sz_cols.append(t[:, :1] + t[:, 1:])
    s_z_t = jnp.transpose(jnp.concatenate(sz_cols, axis=1))

    total = jnp.float32(0.0)
    rows = []
    for g in range(G):
        zi = z_ref[:, g * C:(g + 1) * C]
        w = w_ref[g]
        m2 = jax.lax.dot_general(w + w, zi, (((1,), (1,)), ((), ())))
        d = (s_z_t[g:g + 1, :]
             + jnp.sum(w ** 2, axis=1, keepdims=True)
             - m2)
        d_min = jnp.min(d, axis=0, keepdims=True)
        iota_k = jax.lax.broadcasted_iota(jnp.int32, d.shape, 0)
        idx = jnp.min(jnp.where(d == d_min, iota_k, K), axis=0, keepdims=True)
        rows.append(idx + g * K)
        total = total + jnp.sum(d_min)
    idx_ref[...] = jnp.concatenate(rows, axis=0)
    loss_ref[...] = loss_ref[...] + total


def _sc_gather(table_flat, idx_flat, N, D, G, C):
    info = plsc.get_sparse_core_info()
    nw = info.num_cores * info.num_subcores
    nc = info.num_cores
    rows_w = N // nw
    R = 256
    n_chunks = rows_w // R
    tab_len = table_flat.shape[0]
    mesh = plsc.VectorSubcoreMesh(core_axis_name="c", subcore_axis_name="s")

    @functools.partial(
        pl.kernel,
        mesh=mesh,
        out_type=jax.ShapeDtypeStruct((N, D), jnp.float32),
        scratch_types=[
            pltpu.VMEM((tab_len,), jnp.float32),
            pltpu.VMEM((rows_w * G,), jnp.int32),
            pltpu.VMEM((R, D), jnp.float32),
        ],
        compiler_params=pltpu.CompilerParams(needs_layout_passes=False),
    )
    def gather_k(table_hbm, idx_hbm, out_hbm, table_v, idx_v, out_v):
        wid = lax.axis_index("s") * nc + lax.axis_index("c")
        pltpu.sync_copy(table_hbm, table_v)
        for g in range(G):
            pltpu.sync_copy(idx_hbm.at[pl.ds(g * N + wid * rows_w, rows_w)],
                            idx_v.at[pl.ds(g * rows_w, rows_w)])
        iota = lax.iota(jnp.int32, 16)

        def chunk_body(ch, carry):
            @plsc.parallel_loop(0, R // 16, unroll=4)
            def j_body(j):
                rbase = j * 16
                rows = rbase + iota
                for g in range(G):
                    fk = idx_v[pl.ds(g * rows_w + ch * R + rbase, 16)]
                    addr = fk * C
                    vals = [plsc.load_gather(table_v, [addr + c])
                            for c in range(C)]
                    for c in range(C):
                        col = jnp.full((16,), g * C + c, jnp.int32)
                        plsc.store_scatter(out_v, [rows, col], vals[c])

            pltpu.sync_copy(out_v,
                            out_hbm.at[pl.ds(wid * rows_w + ch * R, R)])
            return carry

        lax.fori_loop(0, n_chunks, chunk_body, 0)

    return gather_k(table_flat, idx_flat)


def _dist_call(z, W, half, off):
    N, D = z.shape
    G, K, C = W.shape
    grid = half // BLOCK
    return pl.pallas_call(
        functools.partial(_dist_kernel, G=G, K=K, C=C),
        grid=(grid,),
        in_specs=[
            pl.BlockSpec((BLOCK, D), lambda i: (i + off, 0)),
            pl.BlockSpec((G, K, C), lambda i: (0, 0, 0)),
        ],
        out_specs=[
            pl.BlockSpec((G, BLOCK), lambda i: (0, i)),
            pl.BlockSpec((1, 1), lambda i: (0, 0)),
        ],
        out_shape=[
            jax.ShapeDtypeStruct((G, half), jnp.int32),
            jax.ShapeDtypeStruct((1, 1), jnp.float32),
        ],
        compiler_params=pltpu.CompilerParams(
            dimension_semantics=("arbitrary",),
        ),
    )(z, W)


def kernel(z, W):
    N, D = z.shape
    G, K, C = W.shape
    half = N // 2
    table = W.reshape(G * K * C)
    idx0, loss0 = _dist_call(z, W, half, 0)
    idx1, loss1 = _dist_call(z, W, half, half // BLOCK)
    zq0 = _sc_gather(table, idx0.reshape(G * half), half, D, G, C)
    zq1 = _sc_gather(table, idx1.reshape(G * half), half, D, G, C)
    zq = jnp.concatenate([zq0, zq1], axis=0)
    loss = (1.0 + BETA) * (loss0[0, 0] + loss1[0, 0]) / (N * D)
    return (zq, loss)


# speedup vs baseline: 5.1397x; 1.0371x over previous
import functools

import jax
import jax.numpy as jnp
from jax import lax
from jax.experimental import pallas as pl
from jax.experimental.pallas import tpu as pltpu
from jax.experimental.pallas import tpu_sc as plsc

BETA = 0.5
BLOCK = 2048


def _dist_kernel(z_ref, w_ref, idx_ref, loss_ref, *, G, K, C):
    @pl.when(pl.program_id(0) == 0)
    def _init():
        loss_ref[...] = jnp.zeros((1, 1), jnp.float32)

    B = z_ref.shape[0]
    sz_cols = []
    for g in range(G):
        zi = z_ref[:, g * C:(g + 1) * C]
        zi2 = zi * zi
        t = zi2[:, :8] + zi2[:, 8:]
        t = t[:, :4] + t[:, 4:]
        t = t[:, :2] + t[:, 2:]
        sz_cols.append(t[:, :1] + t[:, 1:])
    s_z_t = jnp.transpose(jnp.concatenate(sz_cols, axis=1))

    total = jnp.float32(0.0)
    rows = []
    for g in range(G):
        zi = z_ref[:, g * C:(g + 1) * C]
        w = w_ref[g]
        m2 = jax.lax.dot_general(w + w, zi, (((1,), (1,)), ((), ())))
        d = (s_z_t[g:g + 1, :]
             + jnp.sum(w ** 2, axis=1, keepdims=True)
             - m2)
        d_min = jnp.min(d, axis=0, keepdims=True)
        iota_k = jax.lax.broadcasted_iota(jnp.int32, d.shape, 0)
        idx = jnp.min(jnp.where(d == d_min, iota_k, K), axis=0, keepdims=True)
        rows.append(idx + g * K)
        total = total + jnp.sum(d_min)
    idx_ref[...] = jnp.concatenate(rows, axis=0)
    loss_ref[...] = loss_ref[...] + total


def _sc_gather(table_flat, idx_flat, N, D, G, C):
    info = plsc.get_sparse_core_info()
    nw = info.num_cores * info.num_subcores
    nc = info.num_cores
    rows_w = N // nw
    R = 256
    n_chunks = rows_w // R
    tab_len = table_flat.shape[0]
    mesh = plsc.VectorSubcoreMesh(core_axis_name="c", subcore_axis_name="s")

    @functools.partial(
        pl.kernel,
        mesh=mesh,
        out_type=jax.ShapeDtypeStruct((N, D), jnp.float32),
        scratch_types=[
            pltpu.VMEM((tab_len,), jnp.float32),
            pltpu.VMEM((rows_w * G,), jnp.int32),
            pltpu.VMEM((R, D), jnp.float32),
        ],
        compiler_params=pltpu.CompilerParams(needs_layout_passes=False),
    )
    def gather_k(table_hbm, idx_hbm, out_hbm, table_v, idx_v, out_v):
        wid = lax.axis_index("s") * nc + lax.axis_index("c")
        pltpu.sync_copy(table_hbm, table_v)
        for g in range(G):
            pltpu.sync_copy(idx_hbm.at[pl.ds(g * N + wid * rows_w, rows_w)],
                            idx_v.at[pl.ds(g * rows_w, rows_w)])
        iota = lax.iota(jnp.int32, 16)

        def chunk_body(ch, carry):
            @plsc.parallel_loop(0, R // 16, unroll=4)
            def j_body(j):
                rbase = j * 16
                rows = rbase + iota
                for g in range(G):
                    fk = idx_v[pl.ds(g * rows_w + ch * R + rbase, 16)]
                    addr = fk * C
                    vals = [plsc.load_gather(table_v, [addr + c])
                            for c in range(C)]
                    for c in range(C):
                        col = jnp.full((16,), g * C + c, jnp.int32)
                        plsc.store_scatter(out_v, [rows, col], vals[c])

            pltpu.sync_copy(out_v,
                            out_hbm.at[pl.ds(wid * rows_w + ch * R, R)])
            return carry

        lax.fori_loop(0, n_chunks, chunk_body, 0)

    return gather_k(table_flat, idx_flat)


def _dist_call(z, W, half, off):
    N, D = z.shape
    G, K, C = W.shape
    grid = half // BLOCK
    return pl.pallas_call(
        functools.partial(_dist_kernel, G=G, K=K, C=C),
        grid=(grid,),
        in_specs=[
            pl.BlockSpec((BLOCK, D), lambda i: (i + off, 0)),
            pl.BlockSpec((G, K, C), lambda i: (0, 0, 0)),
        ],
        out_specs=[
            pl.BlockSpec((G, BLOCK), lambda i: (0, i)),
            pl.BlockSpec((1, 1), lambda i: (0, 0)),
        ],
        out_shape=[
            jax.ShapeDtypeStruct((G, half), jnp.int32),
            jax.ShapeDtypeStruct((1, 1), jnp.float32),
        ],
        compiler_params=pltpu.CompilerParams(
            dimension_semantics=("arbitrary",),
        ),
    )(z, W)


def kernel(z, W):
    N, D = z.shape
    G, K, C = W.shape
    n_split = 4
    part = N // n_split
    table = W.reshape(G * K * C)
    zqs, losses = [], []
    for p in range(n_split):
        idx_p, loss_p = _dist_call(z, W, part, p * (part // BLOCK))
        zqs.append(_sc_gather(table, idx_p.reshape(G * part), part, D, G, C))
        losses.append(loss_p[0, 0])
    zq = jnp.concatenate(zqs, axis=0)
    loss = (1.0 + BETA) * sum(losses) / (N * D)
    return (zq, loss)


# speedup vs baseline: 5.2958x; 1.0304x over previous
import functools

import jax
import jax.numpy as jnp
from jax import lax
from jax.experimental import pallas as pl
from jax.experimental.pallas import tpu as pltpu
from jax.experimental.pallas import tpu_sc as plsc

BETA = 0.5
BLOCK = 2048


def _dist_kernel(z_ref, w_ref, idx_ref, loss_ref, *, G, K, C):
    @pl.when(pl.program_id(0) == 0)
    def _init():
        loss_ref[...] = jnp.zeros((1, 1), jnp.float32)

    B = z_ref.shape[0]
    sz_cols = []
    for g in range(G):
        zi = z_ref[:, g * C:(g + 1) * C]
        zi2 = zi * zi
        t = zi2[:, :8] + zi2[:, 8:]
        t = t[:, :4] + t[:, 4:]
        t = t[:, :2] + t[:, 2:]
        sz_cols.append(t[:, :1] + t[:, 1:])
    s_z_t = jnp.transpose(jnp.concatenate(sz_cols, axis=1))

    total = jnp.float32(0.0)
    rows = []
    for g in range(G):
        zi = z_ref[:, g * C:(g + 1) * C]
        w = w_ref[g]
        m2 = jax.lax.dot_general(w + w, zi, (((1,), (1,)), ((), ())))
        d = (s_z_t[g:g + 1, :]
             + jnp.sum(w ** 2, axis=1, keepdims=True)
             - m2)
        d_min = jnp.min(d, axis=0, keepdims=True)
        iota_k = jax.lax.broadcasted_iota(jnp.int32, d.shape, 0)
        idx = jnp.min(jnp.where(d == d_min, iota_k, K), axis=0, keepdims=True)
        rows.append(idx + g * K)
        total = total + jnp.sum(d_min)
    idx_ref[...] = jnp.concatenate(rows, axis=0)
    loss_ref[...] = loss_ref[...] + total


def _sc_gather(table_flat, idx_flat, N, D, G, C):
    info = plsc.get_sparse_core_info()
    nw = info.num_cores * info.num_subcores
    nc = info.num_cores
    rows_w = N // nw
    R = 256
    n_chunks = rows_w // R
    tab_len = table_flat.shape[0]
    mesh = plsc.VectorSubcoreMesh(core_axis_name="c", subcore_axis_name="s")

    @functools.partial(
        pl.kernel,
        mesh=mesh,
        out_type=jax.ShapeDtypeStruct((N, D), jnp.float32),
        scratch_types=[
            pltpu.VMEM((tab_len,), jnp.float32),
            pltpu.VMEM((rows_w * G,), jnp.int32),
            pltpu.VMEM((R, D), jnp.float32),
        ],
        compiler_params=pltpu.CompilerParams(needs_layout_passes=False),
    )
    def gather_k(table_hbm, idx_hbm, out_hbm, table_v, idx_v, out_v):
        wid = lax.axis_index("s") * nc + lax.axis_index("c")
        pltpu.sync_copy(table_hbm, table_v)
        for g in range(G):
            pltpu.sync_copy(idx_hbm.at[pl.ds(g * N + wid * rows_w, rows_w)],
                            idx_v.at[pl.ds(g * rows_w, rows_w)])
        iota = lax.iota(jnp.int32, 16)

        def chunk_body(ch, carry):
            @plsc.parallel_loop(0, R // 16, unroll=4)
            def j_body(j):
                rbase = j * 16
                rows = rbase + iota
                for g in range(G):
                    fk = idx_v[pl.ds(g * rows_w + ch * R + rbase, 16)]
                    addr = fk * C
                    vals = [plsc.load_gather(table_v, [addr + c])
                            for c in range(C)]
                    for c in range(C):
                        col = jnp.full((16,), g * C + c, jnp.int32)
                        plsc.store_scatter(out_v, [rows, col], vals[c])

            pltpu.sync_copy(out_v,
                            out_hbm.at[pl.ds(wid * rows_w + ch * R, R)])
            return carry

        lax.fori_loop(0, n_chunks, chunk_body, 0)

    return gather_k(table_flat, idx_flat)


def _dist_call(z, W, half, off):
    N, D = z.shape
    G, K, C = W.shape
    grid = half // BLOCK
    return pl.pallas_call(
        functools.partial(_dist_kernel, G=G, K=K, C=C),
        grid=(grid,),
        in_specs=[
            pl.BlockSpec((BLOCK, D), lambda i: (i + off, 0)),
            pl.BlockSpec((G, K, C), lambda i: (0, 0, 0)),
        ],
        out_specs=[
            pl.BlockSpec((G, BLOCK), lambda i: (0, i)),
            pl.BlockSpec((1, 1), lambda i: (0, 0)),
        ],
        out_shape=[
            jax.ShapeDtypeStruct((G, half), jnp.int32),
            jax.ShapeDtypeStruct((1, 1), jnp.float32),
        ],
        compiler_params=pltpu.CompilerParams(
            dimension_semantics=("arbitrary",),
        ),
    )(z, W)


def kernel(z, W):
    N, D = z.shape
    G, K, C = W.shape
    n_split = 8
    part = N // n_split
    table = W.reshape(G * K * C)
    zqs, losses = [], []
    for p in range(n_split):
        idx_p, loss_p = _dist_call(z, W, part, p * (part // BLOCK))
        zqs.append(_sc_gather(table, idx_p.reshape(G * part), part, D, G, C))
        losses.append(loss_p[0, 0])
    zq = jnp.concatenate(zqs, axis=0)
    loss = (1.0 + BETA) * sum(losses) / (N * D)
    return (zq, loss)


# speedup vs baseline: 5.2967x; 1.0002x over previous
import functools

import jax
import jax.numpy as jnp
from jax import lax
from jax.experimental import pallas as pl
from jax.experimental.pallas import tpu as pltpu
from jax.experimental.pallas import tpu_sc as plsc

BETA = 0.5
BLOCK = 2048


def _dist_kernel(z_ref, w_ref, idx_ref, loss_ref, *, G, K, C):
    @pl.when(pl.program_id(0) == 0)
    def _init():
        loss_ref[...] = jnp.zeros((1, 1), jnp.float32)

    sz_cols = []
    for g in range(G):
        zi = z_ref[:, g * C:(g + 1) * C]
        zi2 = zi * zi
        t = zi2[:, :8] + zi2[:, 8:]
        t = t[:, :4] + t[:, 4:]
        t = t[:, :2] + t[:, 2:]
        sz_cols.append(t[:, :1] + t[:, 1:])
    s_z_t = jnp.transpose(jnp.concatenate(sz_cols, axis=1))

    total = jnp.float32(0.0)
    rows = []
    for g in range(G):
        zi = z_ref[:, g * C:(g + 1) * C]
        w = w_ref[g]
        m2 = jax.lax.dot_general(w + w, zi, (((1,), (1,)), ((), ())))
        d = (s_z_t[g:g + 1, :]
             + jnp.sum(w ** 2, axis=1, keepdims=True)
             - m2)
        d_min = jnp.min(d, axis=0, keepdims=True)
        iota_k = jax.lax.broadcasted_iota(jnp.int32, d.shape, 0)
        idx = jnp.min(jnp.where(d == d_min, iota_k, K), axis=0, keepdims=True)
        rows.append(idx + g * K)
        total = total + jnp.sum(d_min)
    idx_ref[...] = jnp.concatenate(rows, axis=0)
    loss_ref[...] = loss_ref[...] + total


def _sc_gather(table_flat, idx_flat, N, D, G, C):
    info = plsc.get_sparse_core_info()
    nw = info.num_cores * info.num_subcores
    nc = info.num_cores
    rows_w = N // nw
    R = 256
    n_chunks = rows_w // R
    tab_len = table_flat.shape[0]
    mesh = plsc.VectorSubcoreMesh(core_axis_name="c", subcore_axis_name="s")

    @functools.partial(
        pl.kernel,
        mesh=mesh,
        out_type=jax.ShapeDtypeStruct((N, D), jnp.float32),
        scratch_types=[
            pltpu.VMEM((tab_len,), jnp.float32),
            pltpu.VMEM((rows_w * G,), jnp.int32),
            pltpu.VMEM((R, D), jnp.float32),
        ],
        compiler_params=pltpu.CompilerParams(needs_layout_passes=False),
    )
    def gather_k(table_hbm, idx_hbm, out_hbm, table_v, idx_v, out_v):
        wid = lax.axis_index("s") * nc + lax.axis_index("c")
        pltpu.sync_copy(table_hbm, table_v)
        for g in range(G):
            pltpu.sync_copy(idx_hbm.at[pl.ds(g * N + wid * rows_w, rows_w)],
                            idx_v.at[pl.ds(g * rows_w, rows_w)])
        iota = lax.iota(jnp.int32, 16)

        def chunk_body(ch, carry):
            @plsc.parallel_loop(0, R // 16, unroll=4)
            def j_body(j):
                rbase = j * 16
                rows = rbase + iota
                for g in range(G):
                    fk = idx_v[pl.ds(g * rows_w + ch * R + rbase, 16)]
                    addr = fk * C
                    vals = [plsc.load_gather(table_v, [addr + c])
                            for c in range(C)]
                    for c in range(C):
                        col = jnp.full((16,), g * C + c, jnp.int32)
                        plsc.store_scatter(out_v, [rows, col], vals[c])

            pltpu.sync_copy(out_v,
                            out_hbm.at[pl.ds(wid * rows_w + ch * R, R)])
            return carry

        lax.fori_loop(0, n_chunks, chunk_body, 0)

    return gather_k(table_flat, idx_flat)


def _dist_call(z, W, half, off):
    N, D = z.shape
    G, K, C = W.shape
    grid = half // BLOCK
    return pl.pallas_call(
        functools.partial(_dist_kernel, G=G, K=K, C=C),
        grid=(grid,),
        in_specs=[
            pl.BlockSpec((BLOCK, D), lambda i: (i + off, 0)),
            pl.BlockSpec((G, K, C), lambda i: (0, 0, 0)),
        ],
        out_specs=[
            pl.BlockSpec((G, BLOCK), lambda i: (0, i)),
            pl.BlockSpec((1, 1), lambda i: (0, 0)),
        ],
        out_shape=[
            jax.ShapeDtypeStruct((G, half), jnp.int32),
            jax.ShapeDtypeStruct((1, 1), jnp.float32),
        ],
        compiler_params=pltpu.CompilerParams(
            dimension_semantics=("arbitrary",),
        ),
    )(z, W)


def kernel(z, W):
    N, D = z.shape
    G, K, C = W.shape
    n_split = 8
    part = N // n_split
    table = W.reshape(G * K * C)
    zqs, losses = [], []
    for p in range(n_split):
        idx_p, loss_p = _dist_call(z, W, part, p * (part // BLOCK))
        zqs.append(_sc_gather(table, idx_p.reshape(G * part), part, D, G, C))
        losses.append(loss_p[0, 0])
    zq = jnp.concatenate(zqs, axis=0)
    loss = (1.0 + BETA) * sum(losses) / (N * D)
    return (zq, loss)
